# baseline (speedup 1.0000x reference)
import functools

import jax
import jax.numpy as jnp
from jax import lax
from jax.experimental import pallas as pl
from jax.experimental.pallas import tpu as pltpu
from jax.experimental.pallas import tpu_sc as plsc


def kernel(x, where):
    N, C, H, W = x.shape
    OH, OW = 2 * H, 2 * W

    info = plsc.get_sparse_core_info()
    num_cores, num_subcores, L = info.num_cores, info.num_subcores, info.num_lanes
    NW = num_cores * num_subcores

    rows = N * H
    rows_per_w = rows // NW
    RSTEP = 4
    assert rows % NW == 0 and rows_per_w % RSTEP == 0 and H % rows_per_w == 0
    assert C % L == 0
    n_steps = rows_per_w // RSTEP
    n_groups = C // L

    xt = jnp.transpose(x, (0, 2, 3, 1))
    wt = jnp.transpose(where, (0, 2, 3, 1))
    mesh = plsc.VectorSubcoreMesh(core_axis_name="c", subcore_axis_name="s")

    @functools.partial(
        pl.kernel,
        out_type=jax.ShapeDtypeStruct((N, C, OH, OW), jnp.float32),
        mesh=mesh,
        scratch_types=[
            pltpu.VMEM((RSTEP, W, C), jnp.float32),
            pltpu.VMEM((RSTEP, W, C), jnp.int32),
            pltpu.VMEM((L, 8, OW), jnp.float32),
        ],
        compiler_params=pltpu.CompilerParams(
            use_tc_tiling_on_sc=True, needs_layout_passes=False
        ),
    )
    def unpool(x_hbm, w_hbm, out_hbm, x_v, w_v, stage):
        wid = lax.axis_index("s") * num_cores + lax.axis_index("c")
        iota = lax.iota(jnp.int32, L)
        zero = jnp.zeros((L,), jnp.float32)
        zeroi = jnp.zeros((L,), jnp.int32)
        onei = jnp.full((L,), 1, jnp.int32)
        twoi = jnp.full((L,), 2, jnp.int32)
        ihc = [jnp.full((L,), 2 * q, jnp.int32) for q in range(RSTEP)]
        row0 = wid * rows_per_w
        n = row0 // H
        h0 = row0 % H

        for s in range(n_steps):
            hs = h0 + s * RSTEP
            pltpu.sync_copy(x_hbm.at[n, pl.ds(hs, RSTEP)], x_v)
            pltpu.sync_copy(w_hbm.at[n, pl.ds(hs, RSTEP)], w_v)
            tt = hs // RSTEP

            def group_body(cg, carry, _tt=tt):
                c0 = cg * L

                def zero_body(cz, c2):
                    for r in range(8):
                        for v in range(OW // L):
                            stage[cz, r, pl.ds(v * L, L)] = zero
                    return c2

                lax.fori_loop(0, L, zero_body, 0)

                for j in range(W):
                    for ih in range(RSTEP):
                        xv = x_v[ih, j, pl.ds(c0, L)]
                        wv = w_v[ih, j, pl.ds(c0, L)]
                        half = jnp.where(wv >= twoi, onei, zeroi)
                        par = wv - half - half
                        plsc.store_scatter(
                            stage, [iota, half + ihc[ih], par + (2 * j)], xv
                        )

                pltpu.sync_copy(
                    stage,
                    out_hbm.at[n, pl.ds(c0, L), pl.ds(8 * _tt, 8), :],
                )
                return carry

            lax.fori_loop(0, n_groups, group_body, 0)

    return unpool(xt, wt)


# speedup vs baseline: 1.4080x; 1.4080x over previous
import functools

import jax
import jax.numpy as jnp
from jax import lax
from jax.experimental import pallas as pl
from jax.experimental.pallas import tpu as pltpu
from jax.experimental.pallas import tpu_sc as plsc


def kernel(x, where):
    N, C, H, W = x.shape
    OH, OW = 2 * H, 2 * W

    info = plsc.get_sparse_core_info()
    num_cores, num_subcores, L = info.num_cores, info.num_subcores, info.num_lanes
    NW = num_cores * num_subcores

    planes = N * C
    planes_per_w = planes // NW
    PCHUNK = 2
    assert planes % NW == 0
    assert planes_per_w % PCHUNK == 0 and C % planes_per_w == 0
    assert (2 * W) % L == 0
    n_steps = planes_per_w // PCHUNK

    in_plane = H * W
    UPV = 2 * W // L

    xf = x
    wf = where
    mesh = plsc.VectorSubcoreMesh(core_axis_name="c", subcore_axis_name="s")

    @functools.partial(
        pl.kernel,
        out_type=jax.ShapeDtypeStruct((N, C, OH, OW), jnp.float32),
        mesh=mesh,
        scratch_types=[
            pltpu.VMEM((PCHUNK, H, W), jnp.float32),
            pltpu.VMEM((PCHUNK, H, W), jnp.float32),
            pltpu.VMEM((PCHUNK, H, W), jnp.float32),
            pltpu.VMEM((PCHUNK, H, W), jnp.int32),
            pltpu.VMEM((PCHUNK, H, W), jnp.int32),
            pltpu.VMEM((PCHUNK, H, W), jnp.int32),
            pltpu.VMEM((PCHUNK, OH, OW), jnp.float32),
            pltpu.VMEM((PCHUNK, OH, OW), jnp.float32),
            pltpu.SemaphoreType.DMA,
            pltpu.SemaphoreType.DMA,
            pltpu.SemaphoreType.DMA,
            pltpu.SemaphoreType.DMA,
            pltpu.SemaphoreType.DMA,
            pltpu.SemaphoreType.DMA,
            pltpu.SemaphoreType.DMA,
            pltpu.SemaphoreType.DMA,
        ],
        compiler_params=pltpu.CompilerParams(
            use_tc_tiling_on_sc=True, needs_layout_passes=False
        ),
    )
    def unpool(x_hbm, w_hbm, out_hbm,
               x_v0, x_v1, x_v2, w_v0, w_v1, w_v2, o_v0, o_v1,
               sx0, sx1, sx2, sw0, sw1, sw2, so0, so1):
        wid = lax.axis_index("s") * num_cores + lax.axis_index("c")
        iota = lax.iota(jnp.int32, L)
        J = [(iota >> 1) + u * (L // 2) for u in range(UPV)]
        t0 = iota & 1
        t1 = t0 + 2
        zero = jnp.zeros((L,), jnp.float32)
        plane0 = wid * planes_per_w

        xbufs = [(x_v0, sx0), (x_v1, sx1), (x_v2, sx2)]
        wbufs = [(w_v0, sw0), (w_v1, sw1), (w_v2, sw2)]
        obufs = [(o_v0, so0), (o_v1, so1)]

        def start_in(k):
            b3 = k % 3
            p = plane0 + k * PCHUNK
            n = p // C
            c = p % C
            cx = pltpu.async_copy(
                x_hbm.at[n, pl.ds(c, PCHUNK)], xbufs[b3][0], xbufs[b3][1]
            )
            cw = pltpu.async_copy(
                w_hbm.at[n, pl.ds(c, PCHUNK)], wbufs[b3][0], wbufs[b3][1]
            )
            return cx, cw

        def start_out(k):
            b2 = k % 2
            p = plane0 + k * PCHUNK
            n = p // C
            c = p % C
            return pltpu.async_copy(
                obufs[b2][0], out_hbm.at[n, pl.ds(c, PCHUNK)], obufs[b2][1]
            )

        def compute(k):
            x_vb = xbufs[k % 3][0]
            w_vb = wbufs[k % 3][0]
            o_vb = obufs[k % 2][0]

            for q in range(PCHUNK):
                @plsc.parallel_loop(0, H, step=1, unroll=2)
                def _row(i, q=q):
                    x_row = x_vb.at[q, i]
                    w_row = w_vb.at[q, i]
                    for u in range(UPV):
                        xv = plsc.load_gather(x_row, [J[u]])
                        wv = plsc.load_gather(w_row, [J[u]])
                        o_vb[q, 2 * i, pl.ds(u * L, L)] = jnp.where(wv == t0, xv, zero)
                        o_vb[q, 2 * i + 1, pl.ds(u * L, L)] = jnp.where(wv == t1, xv, zero)

        in_d = {0: start_in(0)}
        if n_steps > 1:
            in_d[1] = start_in(1)
        out_d = {}

        def plane_step(k):
            cx, cw = in_d.pop(k)
            cx.wait()
            cw.wait()
            if k + 2 < n_steps:
                in_d[k + 2] = start_in(k + 2)
            if k - 2 >= 0:
                out_d.pop(k - 2).wait()
            compute(k)
            out_d[k] = start_out(k)

        for k in range(n_steps):
            plane_step(k)
        for d in out_d.values():
            d.wait()

    out = unpool(xf, wf)
    return out
